# Initial kernel scaffold; baseline (speedup 1.0000x reference)
import functools

import jax
import jax.numpy as jnp
from jax import lax
from jax.experimental import pallas as pl
from jax.experimental.pallas import tpu as pltpu
from jax.experimental.pallas import tpu_sc as plsc

N = 10000
NP = 10240
D = 128
NE = 160000
DT = 144
SCALE = 1.0 / (128.0 * 128.0)
CLAMP = 85.0
EPS = 1e-6

_NC, _NS = 2, 16
_NW = _NC * _NS
_CH = 100
_CHUNKS = NE // _CH
_WCHUNK = _CHUNKS // _NW

BI, BJ = 512, 1024
NBI, NBJ = NP // BI, NP // BJ
BN = 1024


def _build_sc_segment_sum():
    mesh = plsc.VectorSubcoreMesh(core_axis_name="c", subcore_axis_name="s")

    @functools.partial(
        pl.kernel,
        mesh=mesh,
        out_type=jax.ShapeDtypeStruct((_NC, NP, DT), jnp.float32),
        scratch_types=[
            pltpu.VMEM((_WCHUNK, _CH), jnp.int32),
            pltpu.VMEM((_WCHUNK, _CH), jnp.int32),
            pltpu.VMEM((_CH, DT), jnp.float32),
            pltpu.VMEM_SHARED((NP, DT), jnp.float32),
            pltpu.SemaphoreType.DMA,
        ],
    )
    def sc_seg(h_hbm, edge_hbm, zero_hbm, out_hbm, src_v, dst_v, rows_v, acc, sem):
        cid = lax.axis_index("c")
        sid = lax.axis_index("s")
        wid = sid * _NC + cid
        rps = NP // _NS
        pltpu.sync_copy(zero_hbm.at[pl.ds(sid * rps, rps)],
                        acc.at[pl.ds(sid * rps, rps)])
        plsc.subcore_barrier()
        base = wid * _WCHUNK
        pltpu.sync_copy(edge_hbm.at[0, pl.ds(base, _WCHUNK)], src_v)
        pltpu.sync_copy(edge_hbm.at[1, pl.ds(base, _WCHUNK)], dst_v)

        def body(j, carry):
            pltpu.async_copy(h_hbm.at[src_v.at[j]], rows_v, sem).wait()
            pltpu.sync_copy(rows_v, acc.at[dst_v.at[j]], add=True)
            return carry

        lax.fori_loop(0, _WCHUNK, body, 0)
        plsc.subcore_barrier()
        pltpu.sync_copy(acc.at[pl.ds(sid * rps, rps)],
                        out_hbm.at[cid, pl.ds(sid * rps, rps)])

    return sc_seg


_sc_segment_sum = _build_sc_segment_sum()


def _linear2_body(x_ref, wm_ref, bm_ref, wn_ref, bn_ref, h_ref, y_ref):
    x = x_ref[...]
    h_ref[...] = jnp.maximum(
        jnp.dot(x, wm_ref[...], preferred_element_type=jnp.float32) + bm_ref[...], 0.0)
    y_ref[...] = jnp.maximum(
        jnp.dot(x, wn_ref[...], preferred_element_type=jnp.float32) + bn_ref[...], 0.0)


def _linear2(x, wmT, bm, wnT, bn):
    return pl.pallas_call(
        _linear2_body,
        grid=(NP // BN,),
        in_specs=[
            pl.BlockSpec((BN, D), lambda i: (i, 0)),
            pl.BlockSpec((D, D), lambda i: (0, 0)),
            pl.BlockSpec((1, D), lambda i: (0, 0)),
            pl.BlockSpec((D, D), lambda i: (0, 0)),
            pl.BlockSpec((1, D), lambda i: (0, 0)),
        ],
        out_specs=[pl.BlockSpec((BN, D), lambda i: (i, 0))] * 2,
        out_shape=[jax.ShapeDtypeStruct((NP, D), jnp.float32)] * 2,
    )(x, wmT, bm, wnT, bn)


def _combine_body(y_ref, agg_ref, cnt_ref, p_ref, x_ref, g_ref):
    agg = agg_ref[0] + agg_ref[1]
    x = y_ref[...] + agg / jnp.maximum(cnt_ref[...], 1.0)
    x_ref[...] = x
    nrm = jnp.sqrt(jnp.sum(x * x, axis=1, keepdims=True))
    xn = x / jnp.maximum(nrm, 1e-12)
    g_ref[...] = jnp.dot(xn, p_ref[...], preferred_element_type=jnp.float32)


def _combine(y0, aggv, cnt, p):
    return pl.pallas_call(
        _combine_body,
        grid=(NP // BN,),
        in_specs=[
            pl.BlockSpec((BN, D), lambda i: (i, 0)),
            pl.BlockSpec((2, BN, D), lambda i: (0, i, 0)),
            pl.BlockSpec((BN, 1), lambda i: (i, 0)),
            pl.BlockSpec((D, D), lambda i: (0, 0)),
        ],
        out_specs=[pl.BlockSpec((BN, D), lambda i: (i, 0))] * 2,
        out_shape=[jax.ShapeDtypeStruct((NP, D), jnp.float32)] * 2,
    )(y0, aggv, cnt, p)


def _mat_body(g1_ref, g2_ref, e_ref, r_ref):
    i = pl.program_id(0)
    j = pl.program_id(1)
    a = g1_ref[...].astype(jnp.bfloat16)
    b = g2_ref[...].astype(jnp.bfloat16)
    logits = lax.dot_general(a, b, (((1,), (1,)), ((), ())),
                             preferred_element_type=jnp.float32) * SCALE
    e = jnp.exp(jnp.minimum(logits, CLAMP))
    row = i * BI + lax.broadcasted_iota(jnp.int32, (BI, BJ), 0)
    col = j * BJ + lax.broadcasted_iota(jnp.int32, (BI, BJ), 1)
    e = jnp.where((row < N) & (col < N), e, 0.0)
    e_ref[...] = e.astype(jnp.bfloat16)

    @pl.when(j == 0)
    def _():
        r_ref[...] = jnp.zeros_like(r_ref)

    r_ref[...] += jnp.sum(e, axis=1, keepdims=True)


def _materialize(g1, g2):
    return pl.pallas_call(
        _mat_body,
        grid=(NBI, NBJ),
        in_specs=[
            pl.BlockSpec((BI, D), lambda i, j: (i, 0)),
            pl.BlockSpec((BJ, D), lambda i, j: (j, 0)),
        ],
        out_specs=[
            pl.BlockSpec((BI, BJ), lambda i, j: (i, j)),
            pl.BlockSpec((BI, 1), lambda i, j: (i, 0)),
        ],
        out_shape=[
            jax.ShapeDtypeStruct((NP, NP), jnp.bfloat16),
            jax.ShapeDtypeStruct((NP, 1), jnp.float32),
        ],
    )(g1, g2)


def _rowsum_body(e_ref, v_ref, o_ref, acc_ref):
    j = pl.program_id(1)

    @pl.when(j == 0)
    def _():
        acc_ref[...] = jnp.zeros_like(acc_ref)

    acc_ref[...] += jnp.sum(e_ref[...].astype(jnp.float32) * v_ref[...],
                            axis=1, keepdims=True)

    @pl.when(j == NBJ - 1)
    def _():
        o_ref[...] = acc_ref[...]


def _rowsum(e, vrow):
    return pl.pallas_call(
        _rowsum_body,
        grid=(NBI, NBJ),
        in_specs=[
            pl.BlockSpec((BI, BJ), lambda i, j: (i, j)),
            pl.BlockSpec((1, BJ), lambda i, j: (0, j)),
        ],
        out_specs=pl.BlockSpec((BI, 1), lambda i, j: (i, 0)),
        out_shape=jax.ShapeDtypeStruct((NP, 1), jnp.float32),
        scratch_shapes=[pltpu.VMEM((BI, 1), jnp.float32)],
    )(e, vrow)


def _colsum_body(e_ref, u_ref, o_ref, acc_ref):
    i = pl.program_id(1)

    @pl.when(i == 0)
    def _():
        acc_ref[...] = jnp.zeros_like(acc_ref)

    acc_ref[...] += jnp.sum(e_ref[...].astype(jnp.float32) * u_ref[...],
                            axis=0, keepdims=True)

    @pl.when(i == NBI - 1)
    def _():
        o_ref[...] = acc_ref[...]


def _colsum(e, ucol):
    return pl.pallas_call(
        _colsum_body,
        grid=(NBJ, NBI),
        in_specs=[
            pl.BlockSpec((BI, BJ), lambda j, i: (i, j)),
            pl.BlockSpec((BI, 1), lambda j, i: (i, 0)),
        ],
        out_specs=pl.BlockSpec((1, BJ), lambda j, i: (0, j)),
        out_shape=jax.ShapeDtypeStruct((1, NP), jnp.float32),
        scratch_shapes=[pltpu.VMEM((1, BJ), jnp.float32)],
    )(e, ucol)


def _rowmat_body(e_ref, f_ref, pre_ref, post_ref, o_ref, acc_ref):
    j = pl.program_id(1)

    @pl.when(j == 0)
    def _():
        acc_ref[...] = jnp.zeros_like(acc_ref)

    fb = (pre_ref[...] * f_ref[...]).astype(jnp.bfloat16)
    acc_ref[...] += jnp.dot(e_ref[...], fb, preferred_element_type=jnp.float32)

    @pl.when(j == NBJ - 1)
    def _():
        o_ref[...] = post_ref[...] * acc_ref[...]


def _rowmat(e, f, pre_col, post_col):
    return pl.pallas_call(
        _rowmat_body,
        grid=(NBI, NBJ),
        in_specs=[
            pl.BlockSpec((BI, BJ), lambda i, j: (i, j)),
            pl.BlockSpec((BJ, D), lambda i, j: (j, 0)),
            pl.BlockSpec((BJ, 1), lambda i, j: (j, 0)),
            pl.BlockSpec((BI, 1), lambda i, j: (i, 0)),
        ],
        out_specs=pl.BlockSpec((BI, D), lambda i, j: (i, 0)),
        out_shape=jax.ShapeDtypeStruct((NP, D), jnp.float32),
        scratch_shapes=[pltpu.VMEM((BI, D), jnp.float32)],
    )(e, f, pre_col, post_col)


def _colmat_body(e_ref, f_ref, pre_ref, post_ref, o_ref, acc_ref):
    i = pl.program_id(1)

    @pl.when(i == 0)
    def _():
        acc_ref[...] = jnp.zeros_like(acc_ref)

    fb = (pre_ref[...] * f_ref[...]).astype(jnp.bfloat16)
    acc_ref[...] += lax.dot_general(e_ref[...], fb, (((0,), (0,)), ((), ())),
                                    preferred_element_type=jnp.float32)

    @pl.when(i == NBI - 1)
    def _():
        o_ref[...] = post_ref[...] * acc_ref[...]


def _colmat(e, f, pre_col, post_col):
    return pl.pallas_call(
        _colmat_body,
        grid=(NBJ, NBI),
        in_specs=[
            pl.BlockSpec((BI, BJ), lambda j, i: (i, j)),
            pl.BlockSpec((BI, D), lambda j, i: (i, 0)),
            pl.BlockSpec((BI, 1), lambda j, i: (i, 0)),
            pl.BlockSpec((BJ, 1), lambda j, i: (j, 0)),
        ],
        out_specs=pl.BlockSpec((BJ, D), lambda j, i: (j, 0)),
        out_shape=jax.ShapeDtypeStruct((NP, D), jnp.float32),
        scratch_shapes=[pltpu.VMEM((BJ, D), jnp.float32)],
    )(e, f, pre_col, post_col)


def _cconv_body(x_ref, m_ref, wa_ref, wb_ref, b_ref, o_ref):
    t = jnp.dot(x_ref[...], wa_ref[...], preferred_element_type=jnp.float32)
    t += jnp.dot(m_ref[...], wb_ref[...], preferred_element_type=jnp.float32)
    o_ref[...] = jnp.maximum(t + b_ref[...], 0.0)


def _cconv_lin(x, m, waT, wbT, b):
    return pl.pallas_call(
        _cconv_body,
        grid=(NP // BN,),
        in_specs=[
            pl.BlockSpec((BN, D), lambda i: (i, 0)),
            pl.BlockSpec((BN, D), lambda i: (i, 0)),
            pl.BlockSpec((D, D), lambda i: (0, 0)),
            pl.BlockSpec((D, D), lambda i: (0, 0)),
            pl.BlockSpec((1, D), lambda i: (0, 0)),
        ],
        out_specs=pl.BlockSpec((BN, D), lambda i: (i, 0)),
        out_shape=jax.ShapeDtypeStruct((NP, D), jnp.float32),
    )(x, m, waT, wbT, b)


def _final_body(e_ref, u_ref, v_ref, o_ref):
    o_ref[...] = u_ref[...] * e_ref[...].astype(jnp.float32) * v_ref[...]


def _final(e, ucol, vrow):
    return pl.pallas_call(
        _final_body,
        grid=(NBI, NBJ),
        in_specs=[
            pl.BlockSpec((BI, BJ), lambda i, j: (i, j)),
            pl.BlockSpec((BI, 1), lambda i, j: (i, 0)),
            pl.BlockSpec((1, BJ), lambda i, j: (0, j)),
        ],
        out_specs=pl.BlockSpec((BI, BJ), lambda i, j: (i, j)),
        out_shape=jax.ShapeDtypeStruct((N, N), jnp.float32),
    )(e, ucol, vrow)


def _gconv(x, edge_r, wmT, bm, wnT, bn, p, zero_tbl):
    h, y0 = _linear2(x, wmT, bm, wnT, bn)
    ones_col = jnp.ones((NP, 1), jnp.float32)
    pad_cols = jnp.zeros((NP, DT - D - 1), jnp.float32)
    htbl = jnp.concatenate([h, ones_col, pad_cols], axis=1)
    agg = _sc_segment_sum(htbl, edge_r, zero_tbl)
    aggv = agg[:, :, :D]
    cnt = (agg[0, :, D] + agg[1, :, D]).reshape(NP, 1)
    return _combine(y0, aggv, cnt, p)


def _sinkhorn_uv(g1, g2):
    e, r = _materialize(g1, g2)
    u = 1.0 / jnp.maximum(r, EPS)
    c = _colsum(e, u)
    v = 1.0 / jnp.maximum(c, EPS)
    for _ in range(4):
        r = _rowsum(e, v)
        u = u / jnp.maximum(u * r, EPS)
        c = _colsum(e, u)
        v = v / jnp.maximum(v * c, EPS)
    return e, u, v


def kernel(x1, x2, edge1, edge2, Wm11, bm11, Wn11, bn11, Wm12, bm12, Wn12,
           bn12, A1, Wc1, bc1, Wc2, bc2, Wm21, bm21, Wn21, bn21, Wm22, bm22,
           Wn22, bn22, A2):
    f32 = jnp.float32
    x1p = jnp.pad(x1, ((0, NP - N), (0, 0)))
    x2p = jnp.pad(x2, ((0, NP - N), (0, 0)))
    e1r = edge1.reshape(2, _CHUNKS, _CH)
    e2r = edge2.reshape(2, _CHUNKS, _CH)
    zero_tbl = jnp.zeros((NP, DT), f32)
    eye = jnp.eye(D, dtype=f32)

    def row(b):
        return b.reshape(1, D)

    x1_1, g1 = _gconv(x1p, e1r, Wm11.T, row(bm11), Wn11.T, row(bn11), eye,
                      zero_tbl)
    x2_1, g2 = _gconv(x2p, e2r, Wm12.T, row(bm12), Wn12.T, row(bn12), A1.T,
                      zero_tbl)

    e1, u1, v1 = _sinkhorn_uv(g1, g2)
    v1col = v1.reshape(NP, 1)
    f2m = _rowmat(e1, x2_1, v1col, u1)
    f1m = _colmat(e1, x1_1, u1, v1col)
    x1_2 = _cconv_lin(x1_1, f2m, Wc1[:, :D].T, Wc1[:, D:].T, row(bc1))
    x2_2 = _cconv_lin(x2_1, f1m, Wc2[:, :D].T, Wc2[:, D:].T, row(bc2))

    x1_3, g1b = _gconv(x1_2, e1r, Wm21.T, row(bm21), Wn21.T, row(bn21), eye,
                       zero_tbl)
    x2_3, g2b = _gconv(x2_2, e2r, Wm22.T, row(bm22), Wn22.T, row(bn22), A2.T,
                       zero_tbl)

    e2, u2, v2 = _sinkhorn_uv(g1b, g2b)
    return _final(e2, u2, v2)


# speedup vs baseline: 1.3021x; 1.3021x over previous
import functools

import jax
import jax.numpy as jnp
from jax import lax
from jax.experimental import pallas as pl
from jax.experimental.pallas import tpu as pltpu
from jax.experimental.pallas import tpu_sc as plsc

N = 10000
NP = 10240
D = 128
NE = 160000
DT = 144
SCALE = 1.0 / (128.0 * 128.0)
CLAMP = 85.0
EPS = 1e-6

_NC, _NS = 2, 16
_NW = _NC * _NS
_CH = 100
_CHUNKS = NE // _CH
_WCHUNK = _CHUNKS // _NW

BI, BJ = 512, 1024
NBI, NBJ = NP // BI, NP // BJ
BN = 1024


def _build_sc_segment_sum():
    mesh = plsc.VectorSubcoreMesh(core_axis_name="c", subcore_axis_name="s",
                                  num_cores=_NC, num_subcores=_NS)

    @functools.partial(
        pl.kernel,
        mesh=mesh,
        compiler_params=pltpu.CompilerParams(use_tc_tiling_on_sc=False),
        out_type=jax.ShapeDtypeStruct((_NC, NP, DT), jnp.float32),
        scratch_types=[
            pltpu.VMEM((_WCHUNK, _CH), jnp.int32),
            pltpu.VMEM((_WCHUNK, _CH), jnp.int32),
            pltpu.VMEM((_CH, DT), jnp.float32),
            pltpu.VMEM_SHARED((NP, DT), jnp.float32),
            pltpu.SemaphoreType.DMA,
        ],
    )
    def sc_seg(h_hbm, edge_hbm, zero_hbm, out_hbm, src_v, dst_v, rows_v, acc, sem):
        cid = lax.axis_index("c")
        sid = lax.axis_index("s")
        wid = sid * _NC + cid
        rps = NP // _NS
        pltpu.sync_copy(zero_hbm.at[pl.ds(sid * rps, rps)],
                        acc.at[pl.ds(sid * rps, rps)])
        plsc.subcore_barrier()
        pltpu.sync_copy(edge_hbm.at[0, wid], src_v)
        pltpu.sync_copy(edge_hbm.at[1, wid], dst_v)

        def body(j, carry):
            pltpu.async_copy(h_hbm.at[src_v.at[j]], rows_v, sem).wait()
            pltpu.sync_copy(rows_v, acc.at[dst_v.at[j]], add=True)
            return carry

        lax.fori_loop(0, _WCHUNK, body, 0)
        plsc.subcore_barrier()
        pltpu.sync_copy(acc.at[pl.ds(sid * rps, rps)],
                        out_hbm.at[cid, pl.ds(sid * rps, rps)])

    return sc_seg


_sc_segment_sum = _build_sc_segment_sum()


def _linear2_body(x_ref, wm_ref, bm_ref, wn_ref, bn_ref, h_ref, y_ref):
    x = x_ref[...]
    h_ref[...] = jnp.maximum(
        jnp.dot(x, wm_ref[...], preferred_element_type=jnp.float32) + bm_ref[...], 0.0)
    y_ref[...] = jnp.maximum(
        jnp.dot(x, wn_ref[...], preferred_element_type=jnp.float32) + bn_ref[...], 0.0)


def _linear2(x, wmT, bm, wnT, bn):
    return pl.pallas_call(
        _linear2_body,
        grid=(NP // BN,),
        in_specs=[
            pl.BlockSpec((BN, D), lambda i: (i, 0)),
            pl.BlockSpec((D, D), lambda i: (0, 0)),
            pl.BlockSpec((1, D), lambda i: (0, 0)),
            pl.BlockSpec((D, D), lambda i: (0, 0)),
            pl.BlockSpec((1, D), lambda i: (0, 0)),
        ],
        out_specs=[pl.BlockSpec((BN, D), lambda i: (i, 0))] * 2,
        out_shape=[jax.ShapeDtypeStruct((NP, D), jnp.float32)] * 2,
    )(x, wmT, bm, wnT, bn)


def _combine_body(y_ref, agg_ref, cnt_ref, p_ref, x_ref, g_ref):
    agg = agg_ref[0] + agg_ref[1]
    x = y_ref[...] + agg / jnp.maximum(cnt_ref[...], 1.0)
    x_ref[...] = x
    nrm = jnp.sqrt(jnp.sum(x * x, axis=1, keepdims=True))
    xn = x / jnp.maximum(nrm, 1e-12)
    g_ref[...] = jnp.dot(xn, p_ref[...], preferred_element_type=jnp.float32)


def _combine(y0, aggv, cnt, p):
    return pl.pallas_call(
        _combine_body,
        grid=(NP // BN,),
        in_specs=[
            pl.BlockSpec((BN, D), lambda i: (i, 0)),
            pl.BlockSpec((2, BN, D), lambda i: (0, i, 0)),
            pl.BlockSpec((BN, 1), lambda i: (i, 0)),
            pl.BlockSpec((D, D), lambda i: (0, 0)),
        ],
        out_specs=[pl.BlockSpec((BN, D), lambda i: (i, 0))] * 2,
        out_shape=[jax.ShapeDtypeStruct((NP, D), jnp.float32)] * 2,
    )(y0, aggv, cnt, p)


def _mat_body(g1_ref, g2_ref, e_ref, r_ref):
    i = pl.program_id(0)
    j = pl.program_id(1)
    a = g1_ref[...].astype(jnp.bfloat16)
    b = g2_ref[...].astype(jnp.bfloat16)
    logits = lax.dot_general(a, b, (((1,), (1,)), ((), ())),
                             preferred_element_type=jnp.float32) * SCALE
    e = jnp.exp(jnp.minimum(logits, CLAMP))
    row = i * BI + lax.broadcasted_iota(jnp.int32, (BI, BJ), 0)
    col = j * BJ + lax.broadcasted_iota(jnp.int32, (BI, BJ), 1)
    e = jnp.where((row < N) & (col < N), e, 0.0)
    e_ref[...] = e.astype(jnp.bfloat16)

    @pl.when(j == 0)
    def _():
        r_ref[...] = jnp.zeros_like(r_ref)

    r_ref[...] += jnp.sum(e, axis=1, keepdims=True)


def _materialize(g1, g2):
    return pl.pallas_call(
        _mat_body,
        grid=(NBI, NBJ),
        in_specs=[
            pl.BlockSpec((BI, D), lambda i, j: (i, 0)),
            pl.BlockSpec((BJ, D), lambda i, j: (j, 0)),
        ],
        out_specs=[
            pl.BlockSpec((BI, BJ), lambda i, j: (i, j)),
            pl.BlockSpec((BI, 1), lambda i, j: (i, 0)),
        ],
        out_shape=[
            jax.ShapeDtypeStruct((NP, NP), jnp.bfloat16),
            jax.ShapeDtypeStruct((NP, 1), jnp.float32),
        ],
    )(g1, g2)


def _rowsum_body(e_ref, v_ref, o_ref, acc_ref):
    j = pl.program_id(1)

    @pl.when(j == 0)
    def _():
        acc_ref[...] = jnp.zeros_like(acc_ref)

    acc_ref[...] += jnp.sum(e_ref[...].astype(jnp.float32) * v_ref[...],
                            axis=1, keepdims=True)

    @pl.when(j == NBJ - 1)
    def _():
        o_ref[...] = acc_ref[...]


def _rowsum(e, vrow):
    return pl.pallas_call(
        _rowsum_body,
        grid=(NBI, NBJ),
        in_specs=[
            pl.BlockSpec((BI, BJ), lambda i, j: (i, j)),
            pl.BlockSpec((1, BJ), lambda i, j: (0, j)),
        ],
        out_specs=pl.BlockSpec((BI, 1), lambda i, j: (i, 0)),
        out_shape=jax.ShapeDtypeStruct((NP, 1), jnp.float32),
        scratch_shapes=[pltpu.VMEM((BI, 1), jnp.float32)],
    )(e, vrow)


def _colsum_body(e_ref, u_ref, o_ref, acc_ref):
    i = pl.program_id(1)

    @pl.when(i == 0)
    def _():
        acc_ref[...] = jnp.zeros_like(acc_ref)

    acc_ref[...] += jnp.sum(e_ref[...].astype(jnp.float32) * u_ref[...],
                            axis=0, keepdims=True)

    @pl.when(i == NBI - 1)
    def _():
        o_ref[...] = acc_ref[...]


def _colsum(e, ucol):
    return pl.pallas_call(
        _colsum_body,
        grid=(NBJ, NBI),
        in_specs=[
            pl.BlockSpec((BI, BJ), lambda j, i: (i, j)),
            pl.BlockSpec((BI, 1), lambda j, i: (i, 0)),
        ],
        out_specs=pl.BlockSpec((1, BJ), lambda j, i: (0, j)),
        out_shape=jax.ShapeDtypeStruct((1, NP), jnp.float32),
        scratch_shapes=[pltpu.VMEM((1, BJ), jnp.float32)],
    )(e, ucol)


def _rowmat_body(e_ref, f_ref, pre_ref, post_ref, o_ref, acc_ref):
    j = pl.program_id(1)

    @pl.when(j == 0)
    def _():
        acc_ref[...] = jnp.zeros_like(acc_ref)

    fb = (pre_ref[...] * f_ref[...]).astype(jnp.bfloat16)
    acc_ref[...] += jnp.dot(e_ref[...], fb, preferred_element_type=jnp.float32)

    @pl.when(j == NBJ - 1)
    def _():
        o_ref[...] = post_ref[...] * acc_ref[...]


def _rowmat(e, f, pre_col, post_col):
    return pl.pallas_call(
        _rowmat_body,
        grid=(NBI, NBJ),
        in_specs=[
            pl.BlockSpec((BI, BJ), lambda i, j: (i, j)),
            pl.BlockSpec((BJ, D), lambda i, j: (j, 0)),
            pl.BlockSpec((BJ, 1), lambda i, j: (j, 0)),
            pl.BlockSpec((BI, 1), lambda i, j: (i, 0)),
        ],
        out_specs=pl.BlockSpec((BI, D), lambda i, j: (i, 0)),
        out_shape=jax.ShapeDtypeStruct((NP, D), jnp.float32),
        scratch_shapes=[pltpu.VMEM((BI, D), jnp.float32)],
    )(e, f, pre_col, post_col)


def _colmat_body(e_ref, f_ref, pre_ref, post_ref, o_ref, acc_ref):
    i = pl.program_id(1)

    @pl.when(i == 0)
    def _():
        acc_ref[...] = jnp.zeros_like(acc_ref)

    fb = (pre_ref[...] * f_ref[...]).astype(jnp.bfloat16)
    acc_ref[...] += lax.dot_general(e_ref[...], fb, (((0,), (0,)), ((), ())),
                                    preferred_element_type=jnp.float32)

    @pl.when(i == NBI - 1)
    def _():
        o_ref[...] = post_ref[...] * acc_ref[...]


def _colmat(e, f, pre_col, post_col):
    return pl.pallas_call(
        _colmat_body,
        grid=(NBJ, NBI),
        in_specs=[
            pl.BlockSpec((BI, BJ), lambda j, i: (i, j)),
            pl.BlockSpec((BI, D), lambda j, i: (i, 0)),
            pl.BlockSpec((BI, 1), lambda j, i: (i, 0)),
            pl.BlockSpec((BJ, 1), lambda j, i: (j, 0)),
        ],
        out_specs=pl.BlockSpec((BJ, D), lambda j, i: (j, 0)),
        out_shape=jax.ShapeDtypeStruct((NP, D), jnp.float32),
        scratch_shapes=[pltpu.VMEM((BJ, D), jnp.float32)],
    )(e, f, pre_col, post_col)


def _cconv_body(x_ref, m_ref, wa_ref, wb_ref, b_ref, o_ref):
    t = jnp.dot(x_ref[...], wa_ref[...], preferred_element_type=jnp.float32)
    t += jnp.dot(m_ref[...], wb_ref[...], preferred_element_type=jnp.float32)
    o_ref[...] = jnp.maximum(t + b_ref[...], 0.0)


def _cconv_lin(x, m, waT, wbT, b):
    return pl.pallas_call(
        _cconv_body,
        grid=(NP // BN,),
        in_specs=[
            pl.BlockSpec((BN, D), lambda i: (i, 0)),
            pl.BlockSpec((BN, D), lambda i: (i, 0)),
            pl.BlockSpec((D, D), lambda i: (0, 0)),
            pl.BlockSpec((D, D), lambda i: (0, 0)),
            pl.BlockSpec((1, D), lambda i: (0, 0)),
        ],
        out_specs=pl.BlockSpec((BN, D), lambda i: (i, 0)),
        out_shape=jax.ShapeDtypeStruct((NP, D), jnp.float32),
    )(x, m, waT, wbT, b)


def _final_body(e_ref, u_ref, v_ref, o_ref):
    o_ref[...] = u_ref[...] * e_ref[...].astype(jnp.float32) * v_ref[...]


def _final(e, ucol, vrow):
    return pl.pallas_call(
        _final_body,
        grid=(NBI, NBJ),
        in_specs=[
            pl.BlockSpec((BI, BJ), lambda i, j: (i, j)),
            pl.BlockSpec((BI, 1), lambda i, j: (i, 0)),
            pl.BlockSpec((1, BJ), lambda i, j: (0, j)),
        ],
        out_specs=pl.BlockSpec((BI, BJ), lambda i, j: (i, j)),
        out_shape=jax.ShapeDtypeStruct((N, N), jnp.float32),
    )(e, ucol, vrow)


def _gconv(x, edge_r, wmT, bm, wnT, bn, p, zero_tbl):
    h, y0 = _linear2(x, wmT, bm, wnT, bn)
    ones_col = jnp.ones((NP, 1), jnp.float32)
    pad_cols = jnp.zeros((NP, DT - D - 1), jnp.float32)
    htbl = jnp.concatenate([h, ones_col, pad_cols], axis=1)
    agg = _sc_segment_sum(htbl, edge_r, zero_tbl)
    aggv = agg[:, :, :D]
    cnt = (agg[0, :, D] + agg[1, :, D]).reshape(NP, 1)
    return _combine(y0, aggv, cnt, p)


def _sinkhorn_uv(g1, g2):
    e, r = _materialize(g1, g2)
    u = 1.0 / jnp.maximum(r, EPS)
    c = _colsum(e, u)
    v = 1.0 / jnp.maximum(c, EPS)
    for _ in range(4):
        r = _rowsum(e, v)
        u = u / jnp.maximum(u * r, EPS)
        c = _colsum(e, u)
        v = v / jnp.maximum(v * c, EPS)
    return e, u, v


def kernel(x1, x2, edge1, edge2, Wm11, bm11, Wn11, bn11, Wm12, bm12, Wn12,
           bn12, A1, Wc1, bc1, Wc2, bc2, Wm21, bm21, Wn21, bn21, Wm22, bm22,
           Wn22, bn22, A2):
    f32 = jnp.float32
    x1p = jnp.pad(x1, ((0, NP - N), (0, 0)))
    x2p = jnp.pad(x2, ((0, NP - N), (0, 0)))
    e1r = edge1.reshape(2, _NW, _WCHUNK, _CH)
    e2r = edge2.reshape(2, _NW, _WCHUNK, _CH)
    zero_tbl = jnp.zeros((NP, DT), f32)
    eye = jnp.eye(D, dtype=f32)

    def row(b):
        return b.reshape(1, D)

    x1_1, g1 = _gconv(x1p, e1r, Wm11.T, row(bm11), Wn11.T, row(bn11), eye,
                      zero_tbl)
    x2_1, g2 = _gconv(x2p, e2r, Wm12.T, row(bm12), Wn12.T, row(bn12), A1.T,
                      zero_tbl)

    e1, u1, v1 = _sinkhorn_uv(g1, g2)
    v1col = v1.reshape(NP, 1)
    f2m = _rowmat(e1, x2_1, v1col, u1)
    f1m = _colmat(e1, x1_1, u1, v1col)
    x1_2 = _cconv_lin(x1_1, f2m, Wc1[:, :D].T, Wc1[:, D:].T, row(bc1))
    x2_2 = _cconv_lin(x2_1, f1m, Wc2[:, :D].T, Wc2[:, D:].T, row(bc2))

    x1_3, g1b = _gconv(x1_2, e1r, Wm21.T, row(bm21), Wn21.T, row(bn21), eye,
                       zero_tbl)
    x2_3, g2b = _gconv(x2_2, e2r, Wm22.T, row(bm22), Wn22.T, row(bn22), A2.T,
                       zero_tbl)

    e2, u2, v2 = _sinkhorn_uv(g1b, g2b)
    return _final(e2, u2, v2)
